# baseline (speedup 1.0000x reference)
import functools

import jax
import jax.numpy as jnp
from jax import lax
from jax.experimental import pallas as pl
from jax.experimental.pallas import tpu as pltpu
from jax.experimental.pallas import tpu_sc as plsc

F = 26
V = 1000
K = 32
B = 1024
P = F * (F - 1) // 2
NC, NS = 2, 16
NW = NC * NS
GROUPS = B // 16
RT = V * K // 128


def _sc_body(w2, xiT, w1, interT, emb1T,
             xi_v, a_buf, b_buf, out_v, w1row, sem):
    wid = lax.axis_index("c") * NS + lax.axis_index("s")

    pltpu.sync_copy(xiT, xi_v)

    @pl.when(wid < F)
    def _field():
        f = wid
        pltpu.sync_copy(w1.at[f], w1row)
        for g in range(GROUPS):
            iv = xi_v[f, pl.ds(g * 16, 16)]
            vals = plsc.load_gather(w1row, [iv])
            out_v[pl.ds(g * 16, 16)] = vals
        pltpu.sync_copy(out_v, emb1T.at[f])

    nrem = P % NW
    per = P // NW
    p_start = per * wid + jnp.minimum(wid, nrem)
    count = jnp.where(wid < nrem, per + 1, per)

    def _cond(st):
        rem, _i, rl = st
        return rem >= rl

    def _bdy(st):
        rem, i, rl = st
        return (rem - rl, i + 1, rl - 1)

    rem0, i0, _ = lax.while_loop(_cond, _bdy,
                                 (p_start, jnp.int32(0), jnp.int32(F - 1)))
    j0 = i0 + 1 + rem0

    def _next(i, j):
        jn = j + 1
        wrap = jn == F
        i2 = jnp.where(wrap, i + 1, i)
        j2 = jnp.where(wrap, i2 + 1, jn)
        return i2, j2

    iota16 = lax.iota(jnp.int32, 16)

    def pair_body(n, carry):
        i, j = carry
        ca = pltpu.make_async_copy(w2.at[i, j], a_buf, sem)
        cb = pltpu.make_async_copy(w2.at[j, i], b_buf, sem)
        ca.start()
        cb.start()
        ca.wait()
        cb.wait()

        def g_body(g, _):
            ivi = xi_v[i, pl.ds(g * 16, 16)]
            ivj = xi_v[j, pl.ds(g * 16, 16)]
            res = jnp.zeros((16,), jnp.float32)
            for u in range(16):
                ia = ivi[u]
                ib = ivj[u]
                ra, ca = ia // 4, (ia % 4) * K
                rb, cb = ib // 4, (ib % 4) * K
                a0 = a_buf[ra, pl.ds(ca, 16)]
                a1 = a_buf[ra, pl.ds(ca + 16, 16)]
                b0 = b_buf[rb, pl.ds(cb, 16)]
                b1 = b_buf[rb, pl.ds(cb + 16, 16)]
                r = jnp.sum(a0 * b0 + a1 * b1)
                res = jnp.where(iota16 == u, r, res)
            out_v[pl.ds(g * 16, 16)] = res
            return 0

        lax.fori_loop(0, GROUPS, g_body, 0)
        pltpu.sync_copy(out_v, interT.at[p_start + n])
        return _next(i, j)

    lax.fori_loop(0, count, pair_body, (i0, j0))


@functools.lru_cache(maxsize=1)
def _get_sc_kernel():
    return functools.partial(
        pl.kernel,
        out_type=(jax.ShapeDtypeStruct((P, B), jnp.float32),
                  jax.ShapeDtypeStruct((F, B), jnp.float32)),
        mesh=plsc.VectorSubcoreMesh(core_axis_name="c", subcore_axis_name="s",
                                    num_cores=NC, num_subcores=NS),
        compiler_params=pltpu.CompilerParams(needs_layout_passes=False,
                                             use_tc_tiling_on_sc=False),
        scratch_types=[
            pltpu.VMEM((F, B), jnp.int32),
            pltpu.VMEM((RT, 128), jnp.float32),
            pltpu.VMEM((RT, 128), jnp.float32),
            pltpu.VMEM((B,), jnp.float32),
            pltpu.VMEM((1024,), jnp.float32),
            pltpu.SemaphoreType.DMA,
        ],
    )(_sc_body)


import numpy as _np

_IU, _JU = _np.triu_indices(F, k=1)
_OH_I = _np.eye(F, dtype=_np.float32)[_IU]
_OH_J = _np.eye(F, dtype=_np.float32)[_JU]


def _mlp_body(interT_ref, emb1T_ref, xv_ref, ohi_ref, ohj_ref,
              w1_ref, b1_ref, w2_ref, b2_ref,
              wf_ref, bf_ref, out_ref, label_ref):
    xvT = xv_ref[...].T
    si = jnp.dot(ohi_ref[...], xvT, preferred_element_type=jnp.float32)
    sj = jnp.dot(ohj_ref[...], xvT, preferred_element_type=jnp.float32)
    interT = interT_ref[...] * si * sj
    h1 = jnp.maximum(
        jnp.dot(w1_ref[...], interT, preferred_element_type=jnp.float32)
        + b1_ref[...], 0.0)
    h2 = jnp.maximum(
        jnp.dot(w2_ref[...], h1, preferred_element_type=jnp.float32)
        + b2_ref[...], 0.0)
    featT = jnp.concatenate([emb1T_ref[...] * xvT, h2], axis=0)
    outT = (jnp.dot(wf_ref[...], featT, preferred_element_type=jnp.float32)
            + bf_ref[...])
    out_ref[...] = outT.T
    best = outT[0:1, :]
    bi = jnp.zeros((1, B), jnp.int32)
    for c in range(1, 5):
        v = outT[c:c + 1, :]
        m = v > best
        best = jnp.where(m, v, best)
        bi = jnp.where(m, c, bi)
    label_ref[...] = jnp.reshape(bi, (B,))


_mlp_kernel = pl.pallas_call(
    _mlp_body,
    out_shape=(jax.ShapeDtypeStruct((B, 5), jnp.float32),
               jax.ShapeDtypeStruct((B,), jnp.int32)),
)


def kernel(Xi, Xv, W1, W2, lin1_W, lin1_b, lin2_W, lin2_b, lin_W, lin_b):
    xiT = Xi[:, :, 0].T
    w1p = jnp.pad(W1, ((0, 0), (0, 1024 - V)))
    interT, emb1T = _get_sc_kernel()(W2.reshape(F, F, RT, 128), xiT, w1p)
    out, label = _mlp_kernel(
        interT, emb1T, Xv, jnp.asarray(_OH_I), jnp.asarray(_OH_J),
        lin1_W, lin1_b.reshape(-1, 1),
        lin2_W, lin2_b.reshape(-1, 1), lin_W, lin_b.reshape(-1, 1))
    return (out, label)


# speedup vs baseline: 1.1351x; 1.1351x over previous
import functools

import jax
import jax.numpy as jnp
from jax import lax
from jax.experimental import pallas as pl
from jax.experimental.pallas import tpu as pltpu
from jax.experimental.pallas import tpu_sc as plsc

F = 26
V = 1000
K = 32
B = 1024
P = F * (F - 1) // 2
NC, NS = 2, 16
NW = NC * NS
GROUPS = B // 16
RT = V * K // 128


def _sc_body(w2, xiT, w1, interT, emb1T,
             xi_v, a_buf, b_buf, out_v, w1row, sem):
    wid = lax.axis_index("c") * NS + lax.axis_index("s")

    pltpu.sync_copy(xiT, xi_v)

    @pl.when(wid < F)
    def _field():
        f = wid
        pltpu.sync_copy(w1.at[f], w1row)
        for g in range(GROUPS):
            iv = xi_v[f, pl.ds(g * 16, 16)]
            vals = plsc.load_gather(w1row, [iv])
            out_v[pl.ds(g * 16, 16)] = vals
        pltpu.sync_copy(out_v, emb1T.at[f])

    nrem = P % NW
    per = P // NW
    p_start = per * wid + jnp.minimum(wid, nrem)
    count = jnp.where(wid < nrem, per + 1, per)

    def _cond(st):
        rem, _i, rl = st
        return rem >= rl

    def _bdy(st):
        rem, i, rl = st
        return (rem - rl, i + 1, rl - 1)

    rem0, i0, _ = lax.while_loop(_cond, _bdy,
                                 (p_start, jnp.int32(0), jnp.int32(F - 1)))
    j0 = i0 + 1 + rem0

    def _next(i, j):
        jn = j + 1
        wrap = jn == F
        i2 = jnp.where(wrap, i + 1, i)
        j2 = jnp.where(wrap, i2 + 1, jn)
        return i2, j2

    iota16 = lax.iota(jnp.int32, 16)

    def pair_body(n, carry):
        i, j = carry
        ca = pltpu.make_async_copy(w2.at[i, j], a_buf, sem)
        cb = pltpu.make_async_copy(w2.at[j, i], b_buf, sem)
        ca.start()
        cb.start()
        ca.wait()
        cb.wait()

        def g_body(g, _):
            ivi = xi_v[i, pl.ds(g * 16, 16)]
            ivj = xi_v[j, pl.ds(g * 16, 16)]
            res = jnp.zeros((16,), jnp.float32)
            for u in range(16):
                ia = ivi[u]
                ib = ivj[u]
                a0 = a_buf[ia, pl.ds(0, 16)]
                a1 = a_buf[ia, pl.ds(16, 16)]
                b0 = b_buf[ib, pl.ds(0, 16)]
                b1 = b_buf[ib, pl.ds(16, 16)]
                r = jnp.sum(a0 * b0 + a1 * b1)
                res = jnp.where(iota16 == u, r, res)
            out_v[pl.ds(g * 16, 16)] = res
            return 0

        lax.fori_loop(0, GROUPS, g_body, 0)
        pltpu.sync_copy(out_v, interT.at[p_start + n])
        return _next(i, j)

    lax.fori_loop(0, count, pair_body, (i0, j0))


@functools.lru_cache(maxsize=1)
def _get_sc_kernel():
    return functools.partial(
        pl.kernel,
        out_type=(jax.ShapeDtypeStruct((P, B), jnp.float32),
                  jax.ShapeDtypeStruct((F, B), jnp.float32)),
        mesh=plsc.VectorSubcoreMesh(core_axis_name="c", subcore_axis_name="s",
                                    num_cores=NC, num_subcores=NS),
        compiler_params=pltpu.CompilerParams(needs_layout_passes=False,
                                             use_tc_tiling_on_sc=False),
        scratch_types=[
            pltpu.VMEM((F, B), jnp.int32),
            pltpu.VMEM((V, K), jnp.float32),
            pltpu.VMEM((V, K), jnp.float32),
            pltpu.VMEM((B,), jnp.float32),
            pltpu.VMEM((1024,), jnp.float32),
            pltpu.SemaphoreType.DMA,
        ],
    )(_sc_body)


import numpy as _np

_IU, _JU = _np.triu_indices(F, k=1)
_OH_I = _np.eye(F, dtype=_np.float32)[_IU]
_OH_J = _np.eye(F, dtype=_np.float32)[_JU]


def _mlp_body(interT_ref, emb1T_ref, xv_ref, ohi_ref, ohj_ref,
              w1_ref, b1_ref, w2_ref, b2_ref,
              wf_ref, bf_ref, out_ref, label_ref):
    xvT = xv_ref[...].T
    si = jnp.dot(ohi_ref[...], xvT, preferred_element_type=jnp.float32)
    sj = jnp.dot(ohj_ref[...], xvT, preferred_element_type=jnp.float32)
    interT = interT_ref[...] * si * sj
    h1 = jnp.maximum(
        jnp.dot(w1_ref[...], interT, preferred_element_type=jnp.float32)
        + b1_ref[...], 0.0)
    h2 = jnp.maximum(
        jnp.dot(w2_ref[...], h1, preferred_element_type=jnp.float32)
        + b2_ref[...], 0.0)
    featT = jnp.concatenate([emb1T_ref[...] * xvT, h2], axis=0)
    outT = (jnp.dot(wf_ref[...], featT, preferred_element_type=jnp.float32)
            + bf_ref[...])
    out_ref[...] = outT.T
    best = outT[0:1, :]
    bi = jnp.zeros((1, B), jnp.int32)
    for c in range(1, 5):
        v = outT[c:c + 1, :]
        m = v > best
        best = jnp.where(m, v, best)
        bi = jnp.where(m, c, bi)
    label_ref[...] = jnp.reshape(bi, (B,))


_mlp_kernel = pl.pallas_call(
    _mlp_body,
    out_shape=(jax.ShapeDtypeStruct((B, 5), jnp.float32),
               jax.ShapeDtypeStruct((B,), jnp.int32)),
)


def kernel(Xi, Xv, W1, W2, lin1_W, lin1_b, lin2_W, lin2_b, lin_W, lin_b):
    xiT = Xi[:, :, 0].T
    w1p = jnp.pad(W1, ((0, 0), (0, 1024 - V)))
    interT, emb1T = _get_sc_kernel()(W2, xiT, w1p)
    out, label = _mlp_kernel(
        interT, emb1T, Xv, jnp.asarray(_OH_I), jnp.asarray(_OH_J),
        lin1_W, lin1_b.reshape(-1, 1),
        lin2_W, lin2_b.reshape(-1, 1), lin_W, lin_b.reshape(-1, 1))
    return (out, label)


# speedup vs baseline: 1.1843x; 1.0433x over previous
import functools

import jax
import jax.numpy as jnp
from jax import lax
from jax.experimental import pallas as pl
from jax.experimental.pallas import tpu as pltpu
from jax.experimental.pallas import tpu_sc as plsc

F = 26
V = 1000
K = 32
B = 1024
P = F * (F - 1) // 2
NC, NS = 2, 16
NW = NC * NS
GROUPS = B // 16
RT = V * K // 128


def _sc_body(w2, xiT, w1, interT, emb1T,
             xi_v, a_buf, b_buf, out_v, w1row, sem):
    wid = lax.axis_index("c") * NS + lax.axis_index("s")

    pltpu.sync_copy(xiT, xi_v)

    @pl.when(wid < F)
    def _field():
        f = wid
        pltpu.sync_copy(w1.at[f], w1row)
        for g in range(GROUPS):
            iv = xi_v[f, pl.ds(g * 16, 16)]
            vals = plsc.load_gather(w1row, [iv])
            out_v[pl.ds(g * 16, 16)] = vals
        pltpu.sync_copy(out_v, emb1T.at[f])

    nrem = P % NW
    per = P // NW
    p_start = per * wid + jnp.minimum(wid, nrem)
    count = jnp.where(wid < nrem, per + 1, per)

    def _cond(st):
        rem, _i, rl = st
        return rem >= rl

    def _bdy(st):
        rem, i, rl = st
        return (rem - rl, i + 1, rl - 1)

    rem0, i0, _ = lax.while_loop(_cond, _bdy,
                                 (p_start, jnp.int32(0), jnp.int32(F - 1)))
    j0 = i0 + 1 + rem0

    def _next(i, j):
        jn = j + 1
        wrap = jn == F
        i2 = jnp.where(wrap, i + 1, i)
        j2 = jnp.where(wrap, i2 + 1, jn)
        return i2, j2

    iota16 = lax.iota(jnp.int32, 16)

    def pair_body(n, carry):
        i, j = carry
        ca = pltpu.make_async_copy(w2.at[i, j], a_buf, sem)
        cb = pltpu.make_async_copy(w2.at[j, i], b_buf, sem)
        ca.start()
        cb.start()
        ca.wait()
        cb.wait()

        @plsc.parallel_loop(0, GROUPS)
        def g_body(g):
            ivi = xi_v[i, pl.ds(g * 16, 16)]
            ivj = xi_v[j, pl.ds(g * 16, 16)]
            res = jnp.zeros((16,), jnp.float32)
            for u in range(16):
                ia = ivi[u]
                ib = ivj[u]
                a0 = a_buf[ia, pl.ds(0, 16)]
                a1 = a_buf[ia, pl.ds(16, 16)]
                b0 = b_buf[ib, pl.ds(0, 16)]
                b1 = b_buf[ib, pl.ds(16, 16)]
                r = jnp.sum(a0 * b0 + a1 * b1)
                res = jnp.where(iota16 == u, r, res)
            out_v[pl.ds(g * 16, 16)] = res

        pltpu.sync_copy(out_v, interT.at[p_start + n])
        return _next(i, j)

    lax.fori_loop(0, count, pair_body, (i0, j0))


@functools.lru_cache(maxsize=1)
def _get_sc_kernel():
    return functools.partial(
        pl.kernel,
        out_type=(jax.ShapeDtypeStruct((P, B), jnp.float32),
                  jax.ShapeDtypeStruct((F, B), jnp.float32)),
        mesh=plsc.VectorSubcoreMesh(core_axis_name="c", subcore_axis_name="s",
                                    num_cores=NC, num_subcores=NS),
        compiler_params=pltpu.CompilerParams(needs_layout_passes=False,
                                             use_tc_tiling_on_sc=False),
        scratch_types=[
            pltpu.VMEM((F, B), jnp.int32),
            pltpu.VMEM((V, K), jnp.float32),
            pltpu.VMEM((V, K), jnp.float32),
            pltpu.VMEM((B,), jnp.float32),
            pltpu.VMEM((1024,), jnp.float32),
            pltpu.SemaphoreType.DMA,
        ],
    )(_sc_body)


import numpy as _np

_IU, _JU = _np.triu_indices(F, k=1)
_OH_I = _np.eye(F, dtype=_np.float32)[_IU]
_OH_J = _np.eye(F, dtype=_np.float32)[_JU]


def _mlp_body(interT_ref, emb1T_ref, xv_ref, ohi_ref, ohj_ref,
              w1_ref, b1_ref, w2_ref, b2_ref,
              wf_ref, bf_ref, out_ref, label_ref):
    xvT = xv_ref[...].T
    si = jnp.dot(ohi_ref[...], xvT, preferred_element_type=jnp.float32)
    sj = jnp.dot(ohj_ref[...], xvT, preferred_element_type=jnp.float32)
    interT = interT_ref[...] * si * sj
    h1 = jnp.maximum(
        jnp.dot(w1_ref[...], interT, preferred_element_type=jnp.float32)
        + b1_ref[...], 0.0)
    h2 = jnp.maximum(
        jnp.dot(w2_ref[...], h1, preferred_element_type=jnp.float32)
        + b2_ref[...], 0.0)
    featT = jnp.concatenate([emb1T_ref[...] * xvT, h2], axis=0)
    outT = (jnp.dot(wf_ref[...], featT, preferred_element_type=jnp.float32)
            + bf_ref[...])
    out_ref[...] = outT.T
    best = outT[0:1, :]
    bi = jnp.zeros((1, B), jnp.int32)
    for c in range(1, 5):
        v = outT[c:c + 1, :]
        m = v > best
        best = jnp.where(m, v, best)
        bi = jnp.where(m, c, bi)
    label_ref[...] = jnp.reshape(bi, (B,))


_mlp_kernel = pl.pallas_call(
    _mlp_body,
    out_shape=(jax.ShapeDtypeStruct((B, 5), jnp.float32),
               jax.ShapeDtypeStruct((B,), jnp.int32)),
)


def kernel(Xi, Xv, W1, W2, lin1_W, lin1_b, lin2_W, lin2_b, lin_W, lin_b):
    xiT = Xi[:, :, 0].T
    w1p = jnp.pad(W1, ((0, 0), (0, 1024 - V)))
    interT, emb1T = _get_sc_kernel()(W2, xiT, w1p)
    out, label = _mlp_kernel(
        interT, emb1T, Xv, jnp.asarray(_OH_I), jnp.asarray(_OH_J),
        lin1_W, lin1_b.reshape(-1, 1),
        lin2_W, lin2_b.reshape(-1, 1), lin_W, lin_b.reshape(-1, 1))
    return (out, label)
